# baseline (speedup 1.0000x reference)
import functools

import jax
import jax.numpy as jnp
from jax import lax
from jax.experimental import pallas as pl
from jax.experimental.pallas import tpu as pltpu

S, D, H, E, K = 2048, 768, 768, 8, 2


def _moe_body(x_ref, wr_ref, w1_ref, b1_ref, w2_ref, b2_ref, out_ref, wdense):
    e = pl.program_id(0)

    @pl.when(e == 0)
    def _router():
        xx = x_ref[...]
        logits = lax.dot_general(xx, wr_ref[...], (((1,), (1,)), ((), ())),
                                 preferred_element_type=jnp.float32)
        m = jnp.max(logits, axis=1, keepdims=True)
        ex = jnp.exp(logits - m)
        p = ex / jnp.sum(ex, axis=1, keepdims=True)
        lane = lax.broadcasted_iota(jnp.int32, (S, E), 1)
        m1 = jnp.max(p, axis=1, keepdims=True)
        i1 = jnp.min(jnp.where(p == m1, lane, E), axis=1, keepdims=True)
        p2 = jnp.where(lane == i1, -1.0, p)
        m2 = jnp.max(p2, axis=1, keepdims=True)
        i2 = jnp.min(jnp.where(p2 == m2, lane, E), axis=1, keepdims=True)
        s = m1 + m2
        wdense[...] = jnp.where(lane == i1, m1 / s,
                                jnp.where(lane == i2, m2 / s, 0.0))

    lane = lax.broadcasted_iota(jnp.int32, (S, E), 1)
    w_e = jnp.sum(jnp.where(lane == e, wdense[...], 0.0), axis=1, keepdims=True)
    h = lax.dot_general(x_ref[...], w1_ref[0], (((1,), (1,)), ((), ())),
                        preferred_element_type=jnp.float32)
    h = jnp.maximum(h + b1_ref[0], 0.0)
    y = lax.dot_general(h, w2_ref[0], (((1,), (1,)), ((), ())),
                        preferred_element_type=jnp.float32)
    y = y + b2_ref[0]

    @pl.when(e == 0)
    def _init():
        out_ref[...] = w_e * y

    @pl.when(e > 0)
    def _acc():
        out_ref[...] = out_ref[...] + w_e * y


@jax.jit
def _moe(x2d, Wr, W1, b1, W2, b2):
    return pl.pallas_call(
        _moe_body,
        grid=(E,),
        in_specs=[
            pl.BlockSpec((S, D), lambda e: (0, 0)),
            pl.BlockSpec((E, D), lambda e: (0, 0)),
            pl.BlockSpec((1, H, D), lambda e: (e, 0, 0)),
            pl.BlockSpec((1, 1, H), lambda e: (e, 0, 0)),
            pl.BlockSpec((1, D, H), lambda e: (e, 0, 0)),
            pl.BlockSpec((1, 1, D), lambda e: (e, 0, 0)),
        ],
        out_specs=pl.BlockSpec((S, D), lambda e: (0, 0)),
        out_shape=jax.ShapeDtypeStruct((S, D), jnp.float32),
        scratch_shapes=[pltpu.VMEM((S, E), jnp.float32)],
    )(x2d, Wr, W1, b1.reshape(E, 1, H), W2, b2.reshape(E, 1, D))


def kernel(x, Wr, W1, b1, W2, b2):
    Bs, Ss, Ds = x.shape
    out = _moe(x.reshape(Ss, Ds), Wr, W1, b1, W2, b2)
    return (out.reshape(Bs, Ss, Ds), jnp.float32(0.0))
